# baseline (speedup 1.0000x reference)
import functools

import jax
import jax.numpy as jnp
from jax import lax
from jax.experimental import pallas as pl
from jax.experimental.pallas import tpu as pltpu
from jax.experimental.pallas import tpu_sc as plsc

D_MODEL = 64
_SCALE = 8.0
_CHUNK = 128
_LANES = 16


@functools.lru_cache(maxsize=None)
def _build(B: int):
    info = plsc.get_sparse_core_info()
    nw = info.num_cores * info.num_subcores
    b_per_w = B // nw
    n_chunks = b_per_w // _CHUNK
    mesh = plsc.VectorSubcoreMesh(core_axis_name="c", subcore_axis_name="s")

    @functools.partial(
        pl.kernel,
        mesh=mesh,
        out_type=jax.ShapeDtypeStruct((B, D_MODEL), jnp.float32),
        scratch_types=[
            pltpu.VMEM((n_chunks, _CHUNK), jnp.int32),
            pltpu.VMEM((_CHUNK, D_MODEL), jnp.float32),
            pltpu.SemaphoreType.DMA,
        ],
        compiler_params=pltpu.CompilerParams(use_tc_tiling_on_sc=False),
    )
    def emb_kernel(idx_hbm, table_hbm, out_hbm, idx_v, rows_v, sem):
        wid = lax.axis_index("s") * info.num_cores + lax.axis_index("c")
        base = wid * b_per_w
        pltpu.sync_copy(idx_hbm.at[wid], idx_v)

        def chunk_body(c, carry):
            pltpu.async_copy(table_hbm.at[idx_v.at[c]], rows_v, sem).wait()

            def scale_row(j, carry2):
                for l in range(D_MODEL // _LANES):
                    sl = pl.ds(l * _LANES, _LANES)
                    rows_v[j, sl] = rows_v[j, sl] * _SCALE
                return carry2

            lax.fori_loop(0, _CHUNK, scale_row, 0, unroll=2)
            pltpu.sync_copy(rows_v, out_hbm.at[pl.ds(base + c * _CHUNK, _CHUNK)])
            return carry

        lax.fori_loop(0, n_chunks, chunk_body, 0)

    return emb_kernel


def kernel(x, table):
    s0, s1 = x.shape
    B = s0 * s1
    info = plsc.get_sparse_core_info()
    nw = info.num_cores * info.num_subcores
    idx = x.astype(jnp.int32).reshape(nw, (B // nw) // _CHUNK, _CHUNK)
    out = _build(B)(idx, table)
    return out.reshape(s0, s1, D_MODEL)


# speedup vs baseline: 1.1624x; 1.1624x over previous
import functools

import jax
import jax.numpy as jnp
from jax import lax
from jax.experimental import pallas as pl
from jax.experimental.pallas import tpu as pltpu
from jax.experimental.pallas import tpu_sc as plsc

D_MODEL = 64
_SCALE = 8.0
_CHUNK = 128
_LANES = 16
_NBUF = 8
_K = 4


@functools.lru_cache(maxsize=None)
def _build(B: int):
    info = plsc.get_sparse_core_info()
    nw = info.num_cores * info.num_subcores
    b_per_w = B // nw
    n_chunks = b_per_w // _CHUNK
    n_steady = n_chunks - 2 * _K
    assert n_steady % _NBUF == 0
    mesh = plsc.VectorSubcoreMesh(core_axis_name="c", subcore_axis_name="s")

    @functools.partial(
        pl.kernel,
        mesh=mesh,
        out_type=jax.ShapeDtypeStruct((B, D_MODEL), jnp.float32),
        scratch_types=[
            pltpu.VMEM((n_chunks, _CHUNK), jnp.int32),
            pltpu.VMEM((_NBUF, _CHUNK, D_MODEL), jnp.float32),
        ]
        + [pltpu.SemaphoreType.DMA] * _NBUF,
        compiler_params=pltpu.CompilerParams(use_tc_tiling_on_sc=False),
    )
    def emb_kernel(idx_hbm, table_hbm, out_hbm, idx_v, rows_v, *sems):
        wid = lax.axis_index("s") * info.num_cores + lax.axis_index("c")
        base = wid * b_per_w
        pltpu.sync_copy(idx_hbm.at[wid], idx_v)

        def gather(c, b):
            pltpu.async_copy(table_hbm.at[idx_v.at[c]], rows_v.at[b], sems[b])

        def store(c, b):
            pltpu.async_copy(
                rows_v.at[b], out_hbm.at[pl.ds(base + c * _CHUNK, _CHUNK)], sems[b]
            )

        def wait32(b):
            pltpu.make_async_copy(
                table_hbm.at[pl.ds(0, _CHUNK)], rows_v.at[b], sems[b]
            ).wait()

        def scale(b):
            def scale_row(j, carry):
                for l in range(D_MODEL // _LANES):
                    sl = pl.ds(l * _LANES, _LANES)
                    rows_v[b, j, sl] = rows_v[b, j, sl] * _SCALE
                return carry

            lax.fori_loop(0, _CHUNK, scale_row, 0, unroll=4)

        for b in range(_K):
            gather(b, b)
        for c in range(_K):
            b = c % _NBUF
            wait32(b)
            scale(b)
            store(c, b)
            gather(c + _K, (c + _K) % _NBUF)
        def step(g, carry):
            c0 = _K + g * _NBUF
            for j in range(_NBUF):
                b = (_K + j) % _NBUF
                c = c0 + j
                wait32(b)
                scale(b)
                store(c, b)
                bq = j
                wait32(bq)
                gather(c + _K, bq)
            return carry

        lax.fori_loop(0, n_steady // _NBUF, step, 0)
        for i in range(_K):
            c = n_chunks - _K + i
            b = c % _NBUF
            wait32(b)
            scale(b)
            store(c, b)
        for b in range(_NBUF):
            wait32(b)

    return emb_kernel


def kernel(x, table):
    s0, s1 = x.shape
    B = s0 * s1
    info = plsc.get_sparse_core_info()
    nw = info.num_cores * info.num_subcores
    idx = x.astype(jnp.int32).reshape(nw, (B // nw) // _CHUNK, _CHUNK)
    out = _build(B)(idx, table)
    return out.reshape(s0, s1, D_MODEL)
